# Initial kernel scaffold; baseline (speedup 1.0000x reference)
import jax
import jax.numpy as jnp
from jax.experimental import pallas as pl


def kernel(input, target):
    raise NotImplementedError("write your pallas kernel here")


# speedup vs baseline: 31.0878x; 31.0878x over previous
import functools

import jax
import jax.numpy as jnp
from jax import lax
from jax.experimental import pallas as pl
from jax.experimental.pallas import tpu as pltpu
from jax.experimental.pallas import tpu_sc as plsc

N = 4194304
R = 512
C = N // R
GRID1 = 8
BR = R // GRID1

NC = 2
NS = 16
NW = NC * NS
PW = N // NW
CHUNK = 8192
NCHUNK = PW // CHUNK

NBITS = 15
B = 1 << NBITS
SHIFT = 32 - NBITS - 1

MIN_NEG = 41943


def _k1_bce(x_ref, t_ref, nl_ref, sp_ref, np_ref):
    i = pl.program_id(0)
    x = x_ref[...]
    t = t_ref[...]
    loss = jnp.maximum(x, 0.0) - x * t + jnp.log1p(jnp.exp(-jnp.abs(x)))
    nl_ref[...] = jnp.where(t == 0.0, loss, -1.0)
    ps = jnp.sum(jnp.where(t == 0.0, 0.0, loss))
    npos = jnp.sum(t)

    @pl.when(i == 0)
    def _():
        sp_ref[0, 0] = ps
        np_ref[0, 0] = npos

    @pl.when(i > 0)
    def _():
        sp_ref[0, 0] += ps
        np_ref[0, 0] += npos


def _sc_hist_body(nl_hbm, ocnt_hbm, osum_hbm, buf, hcnt, hsum):
    c = lax.axis_index("c")
    s = lax.axis_index("s")
    wid = s * NC + c
    base = wid * PW

    zeros16 = jnp.zeros((16,), jnp.float32)
    ones16 = jnp.ones((16,), jnp.float32)

    def zbody(j, carry):
        hcnt[pl.ds(j * 16, 16)] = zeros16
        hsum[pl.ds(j * 16, 16)] = zeros16
        return carry

    lax.fori_loop(0, B // 16, zbody, 0)

    def cbody(ci, carry):
        off = pl.multiple_of(base + ci * CHUNK, 8)
        pltpu.sync_copy(nl_hbm.at[pl.ds(off, CHUNK)], buf)

        def ibody(i, icarry):
            v = buf[pl.ds(i * 16, 16)]
            m = v >= 0.0
            bits = plsc.bitcast(v, jnp.int32)
            idx = lax.shift_right_logical(bits, SHIFT)
            idx = jnp.where(m, idx, 0)
            plsc.addupdate_scatter(hcnt, [idx], ones16, mask=m)
            plsc.addupdate_scatter(hsum, [idx], v, mask=m)
            return icarry

        lax.fori_loop(0, CHUNK // 16, ibody, 0)
        return carry

    lax.fori_loop(0, NCHUNK, cbody, 0)

    pltpu.sync_copy(hcnt, ocnt_hbm.at[wid])
    pltpu.sync_copy(hsum, osum_hbm.at[wid])


def _k3_select(cnt_ref, sm_ref, sp_ref, np_ref, out_ref):
    cnt = jnp.sum(cnt_ref[...], axis=0)
    sm = jnp.sum(sm_ref[...], axis=0)

    col = lax.broadcasted_iota(jnp.int32, (128, 128), 0)
    row = lax.broadcasted_iota(jnp.int32, (128, 128), 1)
    upper = (col <= row).astype(jnp.float32)
    incl_row = lax.dot(cnt, upper, precision=lax.Precision.HIGHEST,
                       preferred_element_type=jnp.float32)
    row_tot = incl_row[:, 127:128]
    i2 = lax.broadcasted_iota(jnp.int32, (256, 256), 0)
    j2 = lax.broadcasted_iota(jnp.int32, (256, 256), 1)
    lstrict = (j2 < i2).astype(jnp.float32)
    pref_rows = lax.dot(lstrict, row_tot, precision=lax.Precision.HIGHEST,
                        preferred_element_type=jnp.float32)
    incl = incl_row + pref_rows
    tot = jnp.sum(cnt)
    above = tot - incl

    npos = np_ref[0, 0]
    npi = npos.astype(jnp.int32)
    maxneg = N - npi
    k = jnp.minimum(jnp.maximum(MIN_NEG, 5 * npi), maxneg)
    kf = k.astype(jnp.float32)

    w = jnp.clip((kf - above) / cnt, 0.0, 1.0)
    w = jnp.where(cnt > 0.0, w, 0.0)
    sum_neg = jnp.sum(sm * w)
    count = npos + kf
    out_ref[0, 0] = (sp_ref[0, 0] + sum_neg) / count


def _sc_hist(nl_flat):
    mesh = plsc.VectorSubcoreMesh(core_axis_name="c", subcore_axis_name="s")
    f = pl.kernel(
        _sc_hist_body,
        out_type=[
            jax.ShapeDtypeStruct((NW, B), jnp.float32),
            jax.ShapeDtypeStruct((NW, B), jnp.float32),
        ],
        mesh=mesh,
        compiler_params=pltpu.CompilerParams(needs_layout_passes=False),
        scratch_types=[
            pltpu.VMEM((CHUNK,), jnp.float32),
            pltpu.VMEM((B,), jnp.float32),
            pltpu.VMEM((B,), jnp.float32),
        ],
    )
    return f(nl_flat)


def kernel(input, target):
    x2 = input.reshape(R, C)
    t2 = target.reshape(R, C)
    nl, sp, npos = pl.pallas_call(
        _k1_bce,
        grid=(GRID1,),
        in_specs=[
            pl.BlockSpec((BR, C), lambda i: (i, 0)),
            pl.BlockSpec((BR, C), lambda i: (i, 0)),
        ],
        out_specs=[
            pl.BlockSpec((BR, C), lambda i: (i, 0)),
            pl.BlockSpec(memory_space=pltpu.SMEM),
            pl.BlockSpec(memory_space=pltpu.SMEM),
        ],
        out_shape=[
            jax.ShapeDtypeStruct((R, C), jnp.float32),
            jax.ShapeDtypeStruct((1, 1), jnp.float32),
            jax.ShapeDtypeStruct((1, 1), jnp.float32),
        ],
    )(x2, t2)

    ocnt, osum = _sc_hist(nl.reshape(N))

    out = pl.pallas_call(
        _k3_select,
        in_specs=[
            pl.BlockSpec((NW, 256, 128), lambda: (0, 0, 0)),
            pl.BlockSpec((NW, 256, 128), lambda: (0, 0, 0)),
            pl.BlockSpec(memory_space=pltpu.SMEM),
            pl.BlockSpec(memory_space=pltpu.SMEM),
        ],
        out_specs=pl.BlockSpec(memory_space=pltpu.SMEM),
        out_shape=jax.ShapeDtypeStruct((1, 1), jnp.float32),
    )(ocnt.reshape(NW, 256, 128), osum.reshape(NW, 256, 128), sp, npos)

    return out[0, 0]
